# Initial kernel scaffold; baseline (speedup 1.0000x reference)
import jax
import jax.numpy as jnp
from jax.experimental import pallas as pl


def kernel(x, edge_index, edge_weight, batch, W1, b1, W2, b2, W3, b3, Wf, bf):
    raise NotImplementedError("write your pallas kernel here")


# speedup vs baseline: 6.9202x; 6.9202x over previous
import functools

import jax
import jax.numpy as jnp
from jax import lax
from jax.experimental import pallas as pl
from jax.experimental.pallas import tpu as pltpu
from jax.experimental.pallas import tpu_sc as plsc

N = 50000
E = 800000
F_IN = 128
FH = 64
HALF = 32
G = 128

NC = 2
NS = 16
CH = 128
E_PAD = 802816
STRIPE = 3200
LAST_STRIPE = N - 15 * STRIPE

_MESH = plsc.VectorSubcoreMesh(
    core_axis_name="c", subcore_axis_name="s", num_cores=NC, num_subcores=NS)


@functools.partial(
    pl.kernel,
    out_type=jax.ShapeDtypeStruct((NC * N,), jnp.float32),
    mesh=_MESH,
    scratch_types=[
        pltpu.VMEM((CH,), jnp.int32),
        pltpu.VMEM((CH,), jnp.float32),
        pltpu.VMEM((STRIPE,), jnp.float32),
        pltpu.VMEM_SHARED((N,), jnp.float32),
        pltpu.SemaphoreType.DMA,
    ],
)
def _deg_sc(dst_hbm, w_hbm, out_hbm, dst_v, w_v, zbuf, acc, sem):
    c = lax.axis_index("c")
    s = lax.axis_index("s")
    base = s * STRIPE

    def zero_z(i, carry):
        zbuf[pl.ds(i * 16, 16)] = jnp.zeros((16,), jnp.float32)
        return carry
    lax.fori_loop(0, STRIPE // 16, zero_z, 0)

    @pl.when(s < NS - 1)
    def _():
        pltpu.sync_copy(zbuf, acc.at[pl.ds(base, STRIPE)])

    @pl.when(s == NS - 1)
    def _():
        pltpu.sync_copy(zbuf.at[pl.ds(0, LAST_STRIPE)],
                        acc.at[pl.ds(base, LAST_STRIPE)])

    plsc.subcore_barrier()

    wid = s * NC + c
    epw = E_PAD // (NC * NS)
    ebase = wid * epw

    def chunk(g, carry):
        off = ebase + g * CH
        pltpu.sync_copy(dst_hbm.at[pl.ds(off, CH)], dst_v)
        pltpu.sync_copy(w_hbm.at[pl.ds(off, CH)], w_v)
        pltpu.sync_copy(w_v, acc.at[dst_v], add=True)
        return carry
    lax.fori_loop(0, epw // CH, chunk, 0)

    plsc.subcore_barrier()

    @pl.when(s < NS - 1)
    def _():
        pltpu.sync_copy(acc.at[pl.ds(base, STRIPE)], zbuf)
        pltpu.sync_copy(zbuf, out_hbm.at[pl.ds(c * N + base, STRIPE)])

    @pl.when(s == NS - 1)
    def _():
        pltpu.sync_copy(acc.at[pl.ds(base, LAST_STRIPE)],
                        zbuf.at[pl.ds(0, LAST_STRIPE)])
        pltpu.sync_copy(zbuf.at[pl.ds(0, LAST_STRIPE)],
                        out_hbm.at[pl.ds(c * N + base, LAST_STRIPE)])


@functools.partial(
    pl.kernel,
    out_type=jax.ShapeDtypeStruct((NC, N, HALF), jnp.float32),
    mesh=_MESH,
    scratch_types=[
        pltpu.VMEM((CH,), jnp.int32),
        pltpu.VMEM((CH,), jnp.int32),
        pltpu.VMEM((CH + 16,), jnp.float32),
        pltpu.VMEM((CH, HALF), jnp.float32),
        pltpu.VMEM((400, HALF), jnp.float32),
        pltpu.VMEM_SHARED((N, HALF), jnp.float32),
        pltpu.SemaphoreType.DMA,
    ],
    compiler_params=pltpu.CompilerParams(use_tc_tiling_on_sc=False),
)
def _edge_sc(atab_hbm, src_hbm, dst_hbm, w_hbm, out_hbm,
             src_v, dst_v, w_v, rows_v, zbuf, acc, sem):
    c = lax.axis_index("c")
    s = lax.axis_index("s")
    base = s * STRIPE

    def zero_z(i, carry):
        zbuf[i, pl.ds(0, 16)] = jnp.zeros((16,), jnp.float32)
        zbuf[i, pl.ds(16, 16)] = jnp.zeros((16,), jnp.float32)
        return carry
    lax.fori_loop(0, 400, zero_z, 0)

    @pl.when(s < NS - 1)
    def _():
        for k in range(STRIPE // 400):
            pltpu.sync_copy(zbuf, acc.at[pl.ds(base + k * 400, 400)])

    @pl.when(s == NS - 1)
    def _():
        for k in range(LAST_STRIPE // 400):
            pltpu.sync_copy(zbuf, acc.at[pl.ds(base + k * 400, 400)])

    plsc.subcore_barrier()

    epw = E_PAD // NS
    ebase = s * epw
    coff = c * N

    def chunk(g, carry):
        off = ebase + g * CH
        pltpu.sync_copy(src_hbm.at[pl.ds(off, CH)], src_v)
        pltpu.sync_copy(dst_hbm.at[pl.ds(off, CH)], dst_v)
        pltpu.sync_copy(w_hbm.at[pl.ds(off, CH)], w_v.at[pl.ds(0, CH)])
        for k in range(CH // 16):
            src_v[pl.ds(k * 16, 16)] = src_v[pl.ds(k * 16, 16)] + coff
        pltpu.async_copy(atab_hbm.at[src_v], rows_v, sem).wait()

        def scale(e, carry2):
            ws = w_v[pl.ds(e, 16)][0]
            rows_v[e, pl.ds(0, 16)] = rows_v[e, pl.ds(0, 16)] * ws
            rows_v[e, pl.ds(16, 16)] = rows_v[e, pl.ds(16, 16)] * ws
            return carry2
        lax.fori_loop(0, CH, scale, 0, unroll=4)

        pltpu.sync_copy(rows_v, acc.at[dst_v], add=True)
        return carry
    lax.fori_loop(0, epw // CH, chunk, 0)

    plsc.subcore_barrier()

    @pl.when(s < NS - 1)
    def _():
        pltpu.sync_copy(acc.at[pl.ds(base, STRIPE)],
                        out_hbm.at[c, pl.ds(base, STRIPE)])

    @pl.when(s == NS - 1)
    def _():
        pltpu.sync_copy(acc.at[pl.ds(base, LAST_STRIPE)],
                        out_hbm.at[c, pl.ds(base, LAST_STRIPE)])


B = 2000
R = N // B


def _pre_body(degT_ref, x_ref, w1_ref, dinv_ref, h1_ref, atab_ref):
    deg = jnp.sum(degT_ref[...], axis=1, keepdims=True) + 1.0
    dinv = lax.rsqrt(jnp.maximum(deg, 1e-12))
    h = jnp.dot(x_ref[...], w1_ref[...], preferred_element_type=jnp.float32)
    a = h * dinv
    dinv_ref[...] = dinv
    h1_ref[...] = h
    atab_ref[0] = a[:, :HALF]
    atab_ref[1] = a[:, HALF:]


_pre_tc = pl.pallas_call(
    _pre_body,
    grid=(R,),
    in_specs=[
        pl.BlockSpec((B, 2), lambda j: (j, 0)),
        pl.BlockSpec((B, F_IN), lambda j: (j, 0)),
        pl.BlockSpec((F_IN, FH), lambda j: (0, 0)),
    ],
    out_specs=[
        pl.BlockSpec((B, 1), lambda j: (j, 0)),
        pl.BlockSpec((B, FH), lambda j: (j, 0)),
        pl.BlockSpec((2, B, HALF), lambda j: (0, j, 0)),
    ],
    out_shape=[
        jax.ShapeDtypeStruct((N, 1), jnp.float32),
        jax.ShapeDtypeStruct((N, FH), jnp.float32),
        jax.ShapeDtypeStruct((2, N, HALF), jnp.float32),
    ],
)


def _mid_body(agg_ref, h_ref, dinv_ref, b_ref, w_ref, xl_ref, hn_ref, atab_ref):
    dinv = dinv_ref[...]
    aggc = jnp.concatenate([agg_ref[0], agg_ref[1]], axis=1)
    h = h_ref[...]
    xl = jax.nn.relu(dinv * aggc + (dinv * dinv) * h + b_ref[...])
    hn = jnp.dot(xl, w_ref[...], preferred_element_type=jnp.float32)
    a = hn * dinv
    xl_ref[...] = xl
    hn_ref[...] = hn
    atab_ref[0] = a[:, :HALF]
    atab_ref[1] = a[:, HALF:]


_mid_tc = pl.pallas_call(
    _mid_body,
    grid=(R,),
    in_specs=[
        pl.BlockSpec((2, B, HALF), lambda j: (0, j, 0)),
        pl.BlockSpec((B, FH), lambda j: (j, 0)),
        pl.BlockSpec((B, 1), lambda j: (j, 0)),
        pl.BlockSpec((1, FH), lambda j: (0, 0)),
        pl.BlockSpec((FH, FH), lambda j: (0, 0)),
    ],
    out_specs=[
        pl.BlockSpec((B, FH), lambda j: (j, 0)),
        pl.BlockSpec((B, FH), lambda j: (j, 0)),
        pl.BlockSpec((2, B, HALF), lambda j: (0, j, 0)),
    ],
    out_shape=[
        jax.ShapeDtypeStruct((N, FH), jnp.float32),
        jax.ShapeDtypeStruct((N, FH), jnp.float32),
        jax.ShapeDtypeStruct((2, N, HALF), jnp.float32),
    ],
)


def _pool_body(x1_ref, x2_ref, agg_ref, h3_ref, dinv_ref, b3_ref, batch_ref,
               wf_ref, bf_ref, out_ref, sums, counts):
    j = pl.program_id(0)

    @pl.when(j == 0)
    def _():
        sums[...] = jnp.zeros_like(sums)
        counts[...] = jnp.zeros_like(counts)

    dinv = dinv_ref[...]
    aggc = jnp.concatenate([agg_ref[0], agg_ref[1]], axis=1)
    x3 = jax.nn.relu(dinv * aggc + (dinv * dinv) * h3_ref[...] + b3_ref[...])
    hcat = jnp.concatenate([x1_ref[...], x2_ref[...], x3], axis=1)
    gid = lax.broadcasted_iota(jnp.int32, (B, G), 1)
    m = (batch_ref[...] == gid).astype(jnp.float32)
    sums[...] += lax.dot_general(m, hcat, (((0,), (0,)), ((), ())),
                                 preferred_element_type=jnp.float32)
    counts[...] += lax.dot_general(m, jnp.ones((B, 1), jnp.float32),
                                   (((0,), (0,)), ((), ())),
                                   preferred_element_type=jnp.float32)

    @pl.when(j == R - 1)
    def _():
        pooled = sums[...] / jnp.maximum(counts[...], 1.0)
        logits = jnp.dot(pooled, wf_ref[...],
                         preferred_element_type=jnp.float32) + bf_ref[...]
        zmax = jnp.max(logits, axis=1, keepdims=True)
        ez = jnp.exp(logits - zmax)
        out_ref[...] = ez / jnp.sum(ez, axis=1, keepdims=True)


_pool_tc = pl.pallas_call(
    _pool_body,
    grid=(R,),
    in_specs=[
        pl.BlockSpec((B, FH), lambda j: (j, 0)),
        pl.BlockSpec((B, FH), lambda j: (j, 0)),
        pl.BlockSpec((2, B, HALF), lambda j: (0, j, 0)),
        pl.BlockSpec((B, FH), lambda j: (j, 0)),
        pl.BlockSpec((B, 1), lambda j: (j, 0)),
        pl.BlockSpec((1, FH), lambda j: (0, 0)),
        pl.BlockSpec((B, 1), lambda j: (j, 0)),
        pl.BlockSpec((3 * FH, FH), lambda j: (0, 0)),
        pl.BlockSpec((1, FH), lambda j: (0, 0)),
    ],
    out_specs=pl.BlockSpec((G, FH), lambda j: (0, 0)),
    out_shape=jax.ShapeDtypeStruct((G, FH), jnp.float32),
    scratch_shapes=[
        pltpu.VMEM((G, 3 * FH), jnp.float32),
        pltpu.VMEM((G, 1), jnp.float32),
    ],
)


def kernel(x, edge_index, edge_weight, batch, W1, b1, W2, b2, W3, b3, Wf, bf):
    pad = E_PAD - E
    srcp = jnp.pad(edge_index[0], (0, pad))
    dstp = jnp.pad(edge_index[1], (0, pad))
    wp = jnp.pad(edge_weight, (0, pad))

    degp = _deg_sc(dstp, wp)
    degT = degp.reshape(NC, N).T
    dinv, h1, atab1 = _pre_tc(degT, x, W1)
    agg1 = _edge_sc(atab1.reshape(2 * N, HALF), srcp, dstp, wp)
    x1, h2, atab2 = _mid_tc(agg1, h1, dinv, b1.reshape(1, -1), W2)
    agg2 = _edge_sc(atab2.reshape(2 * N, HALF), srcp, dstp, wp)
    x2, h3, atab3 = _mid_tc(agg2, h2, dinv, b2.reshape(1, -1), W3)
    agg3 = _edge_sc(atab3.reshape(2 * N, HALF), srcp, dstp, wp)
    return _pool_tc(x1, x2, agg3, h3, dinv, b3.reshape(1, -1),
                    batch.reshape(-1, 1), Wf, bf.reshape(1, -1))
